# Initial kernel scaffold; baseline (speedup 1.0000x reference)
import jax
import jax.numpy as jnp
from jax.experimental import pallas as pl


def kernel(x, edge_index, samples, W_gat, a_src, a_dst, W_df, b_df, W_mf, b_mf, W_mlp1, b_mlp1, W_mlp2, b_mlp2, W_pred, b_pred):
    raise NotImplementedError("write your pallas kernel here")


# speedup vs baseline: 12.3505x; 12.3505x over previous
import functools

import jax
import jax.numpy as jnp
from jax import lax
from jax.experimental import pallas as pl
from jax.experimental.pallas import tpu as pltpu
from jax.experimental.pallas import tpu_sc as plsc

N_DIS = 4000
N_MIR = 6000
N = 10000
E = 320000
D_FEAT = 128
F = 64
H = 4
OUT = 64
B = 16384
SLOPE = 0.2

NP = 10240
NC = 2
NS = 16
NW = NC * NS
CH = 128
EP = 327680
ER = EP // CH
RPE = ER // NW
RPT = NP // NS
PAD_DST = N


def _tc1_body(x_ref, wg_ref, asrc_ref, adst_ref,
              wh_ref, es_ref, ed_ref, gms_ref, gmd_ref):
    i = pl.program_id(0)
    x = x_ref[...]
    ms, md = [], []
    for h in range(H):
        wh = jnp.dot(x, wg_ref[h], preferred_element_type=jnp.float32)
        wh_ref[h] = wh
        es = jnp.dot(wh, asrc_ref[h], preferred_element_type=jnp.float32)
        ed = jnp.dot(wh, adst_ref[h], preferred_element_type=jnp.float32)
        es_ref[h, :] = es
        ed_ref[h, :] = ed
        ms.append(jnp.max(es.reshape(8, 128), axis=0))
        md.append(jnp.max(ed.reshape(8, 128), axis=0))
    ps = jnp.stack(ms)
    pd = jnp.stack(md)

    @pl.when(i == 0)
    def _():
        gms_ref[...] = ps
        gmd_ref[...] = pd

    @pl.when(i > 0)
    def _():
        gms_ref[...] = jnp.maximum(gms_ref[...], ps)
        gmd_ref[...] = jnp.maximum(gmd_ref[...], pd)


def _tc1(x, W_gat, a_src, a_dst):
    xp = jnp.zeros((NP, D_FEAT), jnp.float32).at[:N].set(x)
    grid = NP // 1024
    return pl.pallas_call(
        _tc1_body,
        grid=(grid,),
        in_specs=[
            pl.BlockSpec((1024, D_FEAT), lambda i: (i, 0)),
            pl.BlockSpec((H, D_FEAT, F), lambda i: (0, 0, 0)),
            pl.BlockSpec((H, F), lambda i: (0, 0)),
            pl.BlockSpec((H, F), lambda i: (0, 0)),
        ],
        out_specs=[
            pl.BlockSpec((H, 1024, F), lambda i: (0, i, 0)),
            pl.BlockSpec((H, 1024), lambda i: (0, i)),
            pl.BlockSpec((H, 1024), lambda i: (0, i)),
            pl.BlockSpec((H, 128), lambda i: (0, 0)),
            pl.BlockSpec((H, 128), lambda i: (0, 0)),
        ],
        out_shape=[
            jax.ShapeDtypeStruct((H, NP, F), jnp.float32),
            jax.ShapeDtypeStruct((H, NP), jnp.float32),
            jax.ShapeDtypeStruct((H, NP), jnp.float32),
            jax.ShapeDtypeStruct((H, 128), jnp.float32),
            jax.ShapeDtypeStruct((H, 128), jnp.float32),
        ],
    )(xp, W_gat, a_src, a_dst)


def _sca_body(src_hbm, dst_hbm, es_hbm, ed_hbm, gx_hbm,
              ex_hbm, dpart_hbm,
              srcb, dstb, tabs, tabd, exb, gxb, dsrc, cbuf, dcomp,
              dsp0, dsp1, dsp2, dsp3):
    dsps = [dsp0, dsp1, dsp2, dsp3]
    c = lax.axis_index("c")
    s = lax.axis_index("s")
    wid = s * NC + c
    rbase = pl.multiple_of(wid * RPE, RPE)
    pltpu.sync_copy(src_hbm.at[pl.ds(rbase, RPE), :], srcb)
    pltpu.sync_copy(dst_hbm.at[pl.ds(rbase, RPE), :], dstb)
    pltpu.sync_copy(gx_hbm, gxb)

    zv = jnp.zeros((16,), jnp.float32)

    def _zs(i, _):
        dsrc[i, :] = zv
        return 0
    lax.fori_loop(0, CH, _zs, 0)

    def _zc(i, _):
        cbuf[i, :] = zv
        return 0
    lax.fori_loop(0, 1024, _zc, 0)
    for h in range(H):
        @pl.when(s == h)
        def _(h=h):
            for g in range(NP // 1024):
                pltpu.sync_copy(cbuf, dsps[h].at[pl.ds(g * 1024, 1024), :])
    plsc.subcore_barrier()

    lanes = jnp.arange(16, dtype=jnp.int32)
    zl = jnp.zeros((16,), jnp.int32)
    gv = gxb[...]
    for h in range(H):
        pltpu.sync_copy(es_hbm.at[pl.ds(h * NP, N)], tabs)
        pltpu.sync_copy(ed_hbm.at[pl.ds(h * NP, N)], tabd)
        g = gv[h]

        def _erow(j, _, h=h):
            for v in range(CH // 16):
                sl = pl.ds(v * 16, 16)
                sv = srcb[j, sl]
                dv = dstb[j, sl]
                e = plsc.load_gather(tabs, [sv]) + plsc.load_gather(tabd, [dv])
                e = jnp.where(e >= 0.0, e, e * SLOPE)
                exb[j, sl] = jnp.exp(e - g)
            return 0
        lax.fori_loop(0, RPE, _erow, 0)
        pltpu.sync_copy(exb, ex_hbm.at[pl.ds(h * ER + rbase, RPE), :])

        def _srow(j, _, h=h):
            for v in range(CH // 16):
                plsc.store_scatter(
                    dsrc, [jnp.full((16,), v * 16, jnp.int32) + lanes, zl],
                    exb[j, pl.ds(v * 16, 16)])
            pltpu.sync_copy(dsrc, dsps[h].at[dstb.at[j]], add=True)
            return 0
        lax.fori_loop(0, RPE, _srow, 0)

    plsc.subcore_barrier()
    for h in range(H):
        @pl.when(s == h)
        def _(h=h):
            for g in range(NP // 1024):
                pltpu.sync_copy(dsps[h].at[pl.ds(g * 1024, 1024), :], cbuf)

                def _cp(k, _, g=g):
                    k0 = pl.multiple_of(k * 16, 16)
                    dcomp[pl.ds(g * 1024 + k0, 16)] = plsc.load_gather(
                        cbuf, [jnp.full((16,), k0, jnp.int32) + lanes, zl])
                    return 0
                lax.fori_loop(0, 1024 // 16, _cp, 0)
            pltpu.sync_copy(dcomp,
                            dpart_hbm.at[pl.ds(c * (H * NP) + h * NP, NP)])


def _sca(src2, dst2, esf, edf, gmax16):
    mesh = plsc.VectorSubcoreMesh(core_axis_name="c", subcore_axis_name="s")
    return pl.kernel(
        _sca_body,
        out_type=(jax.ShapeDtypeStruct((H * ER, CH), jnp.float32),
                  jax.ShapeDtypeStruct((NC * H * NP,), jnp.float32)),
        mesh=mesh,
        compiler_params=pltpu.CompilerParams(needs_layout_passes=False,
                                             use_tc_tiling_on_sc=False),
        scratch_types=[
            pltpu.VMEM((RPE, CH), jnp.int32),
            pltpu.VMEM((RPE, CH), jnp.int32),
            pltpu.VMEM((N,), jnp.float32),
            pltpu.VMEM((N,), jnp.float32),
            pltpu.VMEM((RPE, CH), jnp.float32),
            pltpu.VMEM((16,), jnp.float32),
            pltpu.VMEM((CH, 16), jnp.float32),
            pltpu.VMEM((1024, 16), jnp.float32),
            pltpu.VMEM((NP,), jnp.float32),
            pltpu.VMEM_SHARED((NP, 16), jnp.float32),
            pltpu.VMEM_SHARED((NP, 16), jnp.float32),
            pltpu.VMEM_SHARED((NP, 16), jnp.float32),
            pltpu.VMEM_SHARED((NP, 16), jnp.float32),
        ],
    )(src2, dst2, esf, edf, gmax16)


def _scb_body(src_hbm, dst_hbm, ex_hbm, dpart_hbm,
              wh0_hbm, wh1_hbm, wh2_hbm, wh3_hbm,
              hpart_hbm,
              srcb, dstb, exb, dtab, tmp, rows, zbuf, attnb,
              hacc, gsem):
    whs = [wh0_hbm, wh1_hbm, wh2_hbm, wh3_hbm]
    c = lax.axis_index("c")
    s = lax.axis_index("s")
    wid = s * NC + c
    rbase = pl.multiple_of(wid * RPE, RPE)
    pltpu.sync_copy(src_hbm.at[pl.ds(rbase, RPE), :], srcb)
    pltpu.sync_copy(dst_hbm.at[pl.ds(rbase, RPE), :], dstb)

    def _zr(i, _):
        for v in range(4):
            zbuf[i, pl.ds(v * 16, 16)] = jnp.zeros((16,), jnp.float32)
        return 0
    lax.fori_loop(0, CH, _zr, 0)

    for h in range(H):
        wh_hbm = whs[h]
        pltpu.sync_copy(ex_hbm.at[pl.ds(h * ER + rbase, RPE), :], exb)
        pltpu.sync_copy(dpart_hbm.at[pl.ds(h * NP, NP)], dtab)
        pltpu.sync_copy(dpart_hbm.at[pl.ds(H * NP + h * NP, NP)], tmp)

        def _dadd(i, _):
            sl = pl.ds(pl.multiple_of(i * 16, 16), 16)
            dtab[sl] = dtab[sl] + tmp[sl]
            return 0
        lax.fori_loop(0, NP // 16, _dadd, 0)

        r0 = pl.multiple_of(s * RPT, CH)
        for k in range(RPT // CH):
            pltpu.sync_copy(zbuf, hacc.at[pl.ds(r0 + k * CH, CH), :])
        plsc.subcore_barrier()

        def _chunk(j, _):
            pltpu.async_copy(wh_hbm.at[srcb.at[j]], rows, gsem).wait()
            for v in range(CH // 16):
                sl = pl.ds(v * 16, 16)
                dv = dstb[j, sl]
                attnb[sl] = exb[j, sl] / (plsc.load_gather(dtab, [dv]) + 1e-9)

            def _rgrp(g, _):
                g0 = pl.multiple_of(g * 16, 16)
                av = attnb[pl.ds(g0, 16)]
                for r in range(16):
                    row = g0 + r
                    a = jnp.full((16,), av[r], jnp.float32)
                    for v in range(4):
                        rows[row, pl.ds(v * 16, 16)] = (
                            rows[row, pl.ds(v * 16, 16)] * a)
                return 0
            lax.fori_loop(0, CH // 16, _rgrp, 0)
            pltpu.sync_copy(rows, hacc.at[dstb.at[j]], add=True)
            return 0
        lax.fori_loop(0, RPE, _chunk, 0)

        plsc.subcore_barrier()
        for k in range(RPT // CH):
            pltpu.sync_copy(hacc.at[pl.ds(r0 + k * CH, CH), :], rows)
            pltpu.sync_copy(rows, hpart_hbm.at[c, h, pl.ds(r0 + k * CH, CH), :])
        plsc.subcore_barrier()


def _scb(src2, dst2, ex1, dpart1, wh0, wh1, wh2, wh3):
    mesh = plsc.VectorSubcoreMesh(core_axis_name="c", subcore_axis_name="s")
    return pl.kernel(
        _scb_body,
        out_type=jax.ShapeDtypeStruct((NC, H, NP, F), jnp.float32),
        mesh=mesh,
        compiler_params=pltpu.CompilerParams(needs_layout_passes=False,
                                             use_tc_tiling_on_sc=False),
        scratch_types=[
            pltpu.VMEM((RPE, CH), jnp.int32),
            pltpu.VMEM((RPE, CH), jnp.int32),
            pltpu.VMEM((RPE, CH), jnp.float32),
            pltpu.VMEM((NP,), jnp.float32),
            pltpu.VMEM((NP,), jnp.float32),
            pltpu.VMEM((CH, F), jnp.float32),
            pltpu.VMEM((CH, F), jnp.float32),
            pltpu.VMEM((CH,), jnp.float32),
            pltpu.VMEM_SHARED((NP, F), jnp.float32),
            pltpu.SemaphoreType.DMA,
        ],
    )(src2, dst2, ex1, dpart1, wh0, wh1, wh2, wh3)


def _tc2_body(hp_ref, w_ref, b_ref, out_ref):
    w = w_ref[0]
    acc = jnp.zeros((1000, OUT), jnp.float32) + b_ref[0, 0][None, :]
    for h in range(H):
        hsum = hp_ref[0, h] + hp_ref[1, h]
        ep = jnp.where(hsum > 0.0, hsum, jnp.exp(hsum) - 1.0)
        acc = acc + jnp.dot(ep, w[F * h:F * h + F],
                            preferred_element_type=jnp.float32)
    out_ref[...] = acc


def _tc2(hpart, W_df, b_df, W_mf, b_mf):
    wst = jnp.stack([W_df, W_mf])
    bst = jnp.stack([b_df, b_mf]).reshape(2, 1, OUT)
    nblk = N_DIS // 1000
    return pl.pallas_call(
        _tc2_body,
        grid=(N // 1000,),
        in_specs=[
            pl.BlockSpec((NC, H, 1000, F), lambda i: (0, 0, i, 0)),
            pl.BlockSpec((1, 2 * D_FEAT, OUT),
                         lambda i: (jnp.where(i < nblk, 0, 1), 0, 0)),
            pl.BlockSpec((1, 1, OUT),
                         lambda i: (jnp.where(i < nblk, 0, 1), 0, 0)),
        ],
        out_specs=pl.BlockSpec((1000, OUT), lambda i: (i, 0)),
        out_shape=jax.ShapeDtypeStruct((N, OUT), jnp.float32),
    )(hpart, wst, bst)


def _scc_body(emb_hbm, midx_hbm, didx_hbm, m_hbm, d_hbm,
              midxb, didxb, rowsm, rowsd, gsem):
    c = lax.axis_index("c")
    s = lax.axis_index("s")
    wid = s * NC + c
    spw = B // NW
    qpw = spw // CH
    rb = pl.multiple_of(wid * qpw, qpw)
    base = pl.multiple_of(wid * spw, spw)
    pltpu.sync_copy(midx_hbm.at[pl.ds(rb, qpw), :], midxb)
    pltpu.sync_copy(didx_hbm.at[pl.ds(rb, qpw), :], didxb)
    for q in range(qpw):
        pltpu.async_copy(emb_hbm.at[midxb.at[q]], rowsm, gsem).wait()
        pltpu.sync_copy(rowsm, m_hbm.at[pl.ds(base + q * CH, CH), :])
        pltpu.async_copy(emb_hbm.at[didxb.at[q]], rowsd, gsem).wait()
        pltpu.sync_copy(rowsd, d_hbm.at[pl.ds(base + q * CH, CH), :])


def _scc(emb, midx2, didx2):
    mesh = plsc.VectorSubcoreMesh(core_axis_name="c", subcore_axis_name="s")
    spw = B // NW
    return pl.kernel(
        _scc_body,
        out_type=(jax.ShapeDtypeStruct((B, OUT), jnp.float32),
                  jax.ShapeDtypeStruct((B, OUT), jnp.float32)),
        mesh=mesh,
        compiler_params=pltpu.CompilerParams(needs_layout_passes=False,
                                             use_tc_tiling_on_sc=False),
        scratch_types=[
            pltpu.VMEM((spw // CH, CH), jnp.int32),
            pltpu.VMEM((spw // CH, CH), jnp.int32),
            pltpu.VMEM((CH, OUT), jnp.float32),
            pltpu.VMEM((CH, OUT), jnp.float32),
            pltpu.SemaphoreType.DMA,
        ],
    )(emb, midx2, didx2)


def _tc3_body(m_ref, d_ref, w1_ref, b1_ref, w2_ref, b2_ref, wp_ref, bp_ref,
              out_ref):
    m = m_ref[...]
    d = d_ref[...]
    gmf = m * d
    z = jnp.dot(m, w1_ref[:OUT], preferred_element_type=jnp.float32)
    z = z + jnp.dot(d, w1_ref[OUT:], preferred_element_type=jnp.float32)
    z = jnp.maximum(z + b1_ref[0][None, :], 0.0)
    z = jnp.dot(z, w2_ref[...], preferred_element_type=jnp.float32)
    z = jnp.maximum(z + b2_ref[0][None, :], 0.0)
    logit = jnp.dot(gmf, wp_ref[:OUT], preferred_element_type=jnp.float32)
    logit = logit + jnp.dot(z, wp_ref[OUT:], preferred_element_type=jnp.float32)
    logit = logit[:, 0] + bp_ref[0, 0]
    out_ref[0, 0, :] = jax.nn.sigmoid(logit)


def _tc3(m_emb, d_emb, W_mlp1, b_mlp1, W_mlp2, b_mlp2, W_pred, b_pred):
    bn = 2048
    out = pl.pallas_call(
        _tc3_body,
        grid=(B // bn,),
        in_specs=[
            pl.BlockSpec((bn, OUT), lambda i: (i, 0)),
            pl.BlockSpec((bn, OUT), lambda i: (i, 0)),
            pl.BlockSpec((2 * OUT, 64), lambda i: (0, 0)),
            pl.BlockSpec((1, 64), lambda i: (0, 0)),
            pl.BlockSpec((64, 32), lambda i: (0, 0)),
            pl.BlockSpec((1, 32), lambda i: (0, 0)),
            pl.BlockSpec((OUT + 32, 1), lambda i: (0, 0)),
            pl.BlockSpec((1, 1), lambda i: (0, 0)),
        ],
        out_specs=pl.BlockSpec((1, 1, bn), lambda i: (i, 0, 0)),
        out_shape=jax.ShapeDtypeStruct((B // bn, 1, bn), jnp.float32),
    )(m_emb, d_emb, W_mlp1, b_mlp1.reshape(1, 64), W_mlp2,
      b_mlp2.reshape(1, 32), W_pred, b_pred.reshape(1, 1))
    return out.reshape(B)


def kernel(x, edge_index, samples, W_gat, a_src, a_dst, W_df, b_df, W_mf, b_mf,
           W_mlp1, b_mlp1, W_mlp2, b_mlp2, W_pred, b_pred):
    edge_index = edge_index.astype(jnp.int32)
    samples = samples.astype(jnp.int32)
    src2 = jnp.pad(edge_index[0], (0, EP - E)).reshape(ER, CH)
    dst2 = jnp.pad(edge_index[1], (0, EP - E),
                   constant_values=PAD_DST).reshape(ER, CH)
    wh, e_src, e_dst, gms, gmd = _tc1(x, W_gat, a_src, a_dst)
    gmax = jnp.max(gms, axis=1) + jnp.max(gmd, axis=1)
    gmax = jnp.where(gmax >= 0.0, gmax, gmax * SLOPE)
    gmax16 = jnp.zeros((16,), jnp.float32).at[:H].set(gmax)
    ex1, dpart1 = _sca(src2, dst2, e_src.reshape(H * NP),
                       e_dst.reshape(H * NP), gmax16)
    hpart = _scb(src2, dst2, ex1, dpart1, wh[0, :N], wh[1, :N], wh[2, :N],
                 wh[3, :N])
    emb = _tc2(hpart, W_df, b_df, W_mf, b_mf)
    midx2 = (jnp.clip(samples[:, 0] - 1, 0, N_MIR - 1) + N_DIS).reshape(
        B // CH, CH)
    didx2 = jnp.clip(samples[:, 1] - 1, 0, N_DIS - 1).reshape(B // CH, CH)
    m_emb, d_emb = _scc(emb, midx2, didx2)
    return _tc3(m_emb, d_emb, W_mlp1, b_mlp1, W_mlp2, b_mlp2, W_pred, b_pred)
